# baseline (speedup 1.0000x reference)
import functools

import jax
import jax.numpy as jnp
from jax import lax
from jax.experimental import pallas as pl
from jax.experimental.pallas import tpu as pltpu
from jax.experimental.pallas import tpu_sc as plsc

_H = 14
_F = 64
_FO = _F + _H - 1
_CHUNK = 256

_GATHER_DNUMS = lax.GatherDimensionNumbers(
    offset_dims=(), collapsed_slice_dims=(0,), start_index_map=(0,))


def _gather16(v, idx):
    return lax.gather(v, idx[:, None], _GATHER_DNUMS, slice_sizes=(1,),
                      mode=lax.GatherScatterMode.PROMISE_IN_BOUNDS)


def _make_row_body(in_v, out_v, lane, lane_f, cidx):
    def row_body(r, carry):
        v0 = in_v[r, pl.ds(0, 16)]
        a = in_v[r, pl.ds(3, 16)]
        b = in_v[r, pl.ds(19, 16)]
        c = in_v[r, pl.ds(35, 16)]
        d = in_v[r, pl.ds(48, 16)]
        t = _gather16(v0, cidx)
        oh = jnp.where(t == lane_f,
                       jnp.asarray(1.0, jnp.float32),
                       jnp.asarray(0.0, jnp.float32))
        w0 = jnp.where(lane < _H, oh, t)
        out_v[r, pl.ds(0, 16)] = w0
        out_v[r, pl.ds(16, 16)] = a
        out_v[r, pl.ds(32, 16)] = b
        out_v[r, pl.ds(48, 16)] = c
        out_v[r, pl.ds(61, 16)] = d
        return carry

    return row_body


def _make_sc_kernel(B, L):
    rows = B * L
    info = plsc.get_sparse_core_info()
    nc, ns = info.num_cores, info.num_subcores
    nw = nc * ns
    rows_per_w = rows // nw
    n_pairs = rows_per_w // (2 * _CHUNK)
    mesh = plsc.VectorSubcoreMesh(core_axis_name="c", subcore_axis_name="s")

    @functools.partial(
        pl.kernel,
        mesh=mesh,
        out_type=jax.ShapeDtypeStruct((rows, _FO), jnp.float32),
        scratch_types=[
            pltpu.VMEM((_CHUNK, _F), jnp.float32),
            pltpu.VMEM((_CHUNK, _F), jnp.float32),
            pltpu.VMEM((_CHUNK, _FO), jnp.float32),
            pltpu.VMEM((_CHUNK, _FO), jnp.float32),
            pltpu.SemaphoreType.DMA,
            pltpu.SemaphoreType.DMA,
            pltpu.SemaphoreType.DMA,
            pltpu.SemaphoreType.DMA,
        ],
    )
    def sc_kernel(in_hbm3, out_hbm, in_v0, in_v1, out_v0, out_v1,
                  si0, si1, so0, so1):
        in_hbm = in_hbm3.reshape(B * L, _F)
        wid = lax.axis_index("s") * nc + lax.axis_index("c")
        w_base = wid * rows_per_w

        lane = lax.broadcasted_iota(jnp.int32, (16,), 0)
        lane_f = (((lane + 15) % 16) + 1).astype(jnp.float32)
        cidx = jnp.where(lane < _H, 0, lane - (_H - 1))

        pltpu.async_copy(in_hbm.at[pl.ds(w_base, _CHUNK)], in_v0, si0)

        def phase(p, c, in_v, out_v, si, si_next, so, in_v_next):
            base = w_base + c * _CHUNK
            pltpu.make_async_copy(
                in_hbm.at[pl.ds(base, _CHUNK)], in_v, si).wait()
            @pl.when(c + 1 < 2 * n_pairs)
            def _():
                pltpu.async_copy(
                    in_hbm.at[pl.ds(base + _CHUNK, _CHUNK)], in_v_next,
                    si_next)
            @pl.when(p > 0)
            def _():
                pltpu.make_async_copy(
                    out_v, out_hbm.at[pl.ds(base, _CHUNK)], so).wait()
            plsc.parallel_loop(0, _CHUNK, 1, unroll=8, carry=jnp.int32(0))(
                _make_row_body(in_v, out_v, lane, lane_f, cidx))
            pltpu.async_copy(out_v, out_hbm.at[pl.ds(base, _CHUNK)], so)

        def pair_body(p, _):
            phase(p, 2 * p, in_v0, out_v0, si0, si1, so0, in_v1)
            phase(p, 2 * p + 1, in_v1, out_v1, si1, si0, so1, in_v0)
            return 0

        lax.fori_loop(0, n_pairs, pair_body, 0)
        pltpu.make_async_copy(
            out_v0, out_hbm.at[pl.ds(w_base, _CHUNK)], so0).wait()
        pltpu.make_async_copy(
            out_v1, out_hbm.at[pl.ds(w_base, _CHUNK)], so1).wait()

    return sc_kernel


def kernel(seq):
    B, L, F = seq.shape
    out = _make_sc_kernel(B, L)(seq)
    return out.reshape(B, L, _FO)


# speedup vs baseline: 1.1841x; 1.1841x over previous
import functools

import jax
import jax.numpy as jnp
from jax import lax
from jax.experimental import pallas as pl
from jax.experimental.pallas import tpu as pltpu
from jax.experimental.pallas import tpu_sc as plsc

_H = 14
_F = 64
_FO = _F + _H - 1
_CHUNK = 256

_GATHER_DNUMS = lax.GatherDimensionNumbers(
    offset_dims=(), collapsed_slice_dims=(0,), start_index_map=(0,))


def _gather16(v, idx):
    return lax.gather(v, idx[:, None], _GATHER_DNUMS, slice_sizes=(1,),
                      mode=lax.GatherScatterMode.PROMISE_IN_BOUNDS)


def _make_row_body(in_v, out_v, lane, lane_f, cidx):
    def row_body(r, carry):
        v0 = in_v[r, pl.ds(0, 16)]
        a = in_v[r, pl.ds(3, 16)]
        b = in_v[r, pl.ds(19, 16)]
        c = in_v[r, pl.ds(35, 16)]
        d = in_v[r, pl.ds(48, 16)]
        t = _gather16(v0, cidx)
        oh = jnp.where(t == lane_f,
                       jnp.asarray(1.0, jnp.float32),
                       jnp.asarray(0.0, jnp.float32))
        w0 = jnp.where(lane < _H, oh, t)
        out_v[r, pl.ds(0, 16)] = w0
        out_v[r, pl.ds(16, 16)] = a
        out_v[r, pl.ds(32, 16)] = b
        out_v[r, pl.ds(48, 16)] = c
        out_v[r, pl.ds(61, 16)] = d
        return carry

    return row_body


def _make_sc_kernel(rows):
    info = plsc.get_sparse_core_info()
    nc, ns = info.num_cores, info.num_subcores
    nw = nc * ns
    rows_per_w = rows // nw
    n_pairs = rows_per_w // (2 * _CHUNK)
    mesh = plsc.VectorSubcoreMesh(core_axis_name="c", subcore_axis_name="s")

    @functools.partial(
        pl.kernel,
        mesh=mesh,
        out_type=jax.ShapeDtypeStruct((rows, _FO), jnp.float32),
        scratch_types=[
            pltpu.VMEM((_CHUNK, _F), jnp.float32),
            pltpu.VMEM((_CHUNK, _F), jnp.float32),
            pltpu.VMEM((_CHUNK, _FO), jnp.float32),
            pltpu.VMEM((_CHUNK, _FO), jnp.float32),
            pltpu.SemaphoreType.DMA,
            pltpu.SemaphoreType.DMA,
            pltpu.SemaphoreType.DMA,
            pltpu.SemaphoreType.DMA,
        ],
    )
    def sc_kernel(in_hbm, out_hbm, in_v0, in_v1, out_v0, out_v1,
                  si0, si1, so0, so1):
        wid = lax.axis_index("s") * nc + lax.axis_index("c")
        w_base = wid * rows_per_w

        lane = lax.broadcasted_iota(jnp.int32, (16,), 0)
        lane_f = (((lane + 15) % 16) + 1).astype(jnp.float32)
        cidx = jnp.where(lane < _H, 0, lane - (_H - 1))

        pltpu.async_copy(in_hbm.at[pl.ds(w_base, _CHUNK)], in_v0, si0)

        def phase(p, c, in_v, out_v, si, si_next, so, in_v_next):
            base = w_base + c * _CHUNK
            pltpu.make_async_copy(
                in_hbm.at[pl.ds(base, _CHUNK)], in_v, si).wait()
            @pl.when(c + 1 < 2 * n_pairs)
            def _():
                pltpu.async_copy(
                    in_hbm.at[pl.ds(base + _CHUNK, _CHUNK)], in_v_next,
                    si_next)
            @pl.when(p > 0)
            def _():
                pltpu.make_async_copy(
                    out_v, out_hbm.at[pl.ds(base, _CHUNK)], so).wait()
            plsc.parallel_loop(0, _CHUNK, 1, unroll=16, carry=jnp.int32(0))(
                _make_row_body(in_v, out_v, lane, lane_f, cidx))
            pltpu.async_copy(out_v, out_hbm.at[pl.ds(base, _CHUNK)], so)

        def pair_body(p, _):
            phase(p, 2 * p, in_v0, out_v0, si0, si1, so0, in_v1)
            phase(p, 2 * p + 1, in_v1, out_v1, si1, si0, so1, in_v0)
            return 0

        lax.fori_loop(0, n_pairs, pair_body, 0)
        pltpu.make_async_copy(
            out_v0, out_hbm.at[pl.ds(w_base, _CHUNK)], so0).wait()
        pltpu.make_async_copy(
            out_v1, out_hbm.at[pl.ds(w_base, _CHUNK)], so1).wait()

    return sc_kernel


def kernel(seq):
    B, L, F = seq.shape
    rows = B * L
    flat = seq.reshape(rows, F)
    out = _make_sc_kernel(rows)(flat)
    return out.reshape(B, L, _FO)
